# baseline (speedup 1.0000x reference)
import functools
from math import sqrt

import numpy as np
import jax
import jax.numpy as jnp
from jax import lax
from jax.experimental import pallas as pl
from jax.experimental.pallas import tpu as pltpu

_FACTOR = 5
_NEG = -3.0e38

_CONST_CACHE = {}


def _sample_count_matrix(L_Q, L_K, U_part):
    ck = (L_Q, L_K, U_part)
    if ck not in _CONST_CACHE:
        with jax.ensure_compile_time_eval():
            skey = jax.random.key(42)
            idx = jax.random.randint(skey, (L_Q, U_part), 0, L_K)
        idx_np = np.asarray(jax.device_get(idx)).astype(np.int64)
        w = np.zeros((L_Q, L_K), dtype=np.int8)
        np.add.at(w, (np.arange(L_Q)[:, None], idx_np), 1)
        _CONST_CACHE[ck] = w
    return _CONST_CACHE[ck]


def _m_scores_body(q_ref, k_ref, w_ref, m_ref, *, L, D, CH):
    k = k_ref[0]
    for c in range(L // CH):
        sl = pl.ds(c * CH, CH)
        q = q_ref[0, sl, :]
        s = lax.dot_general(q, k, (((1,), (1,)), ((), ())),
                            preferred_element_type=jnp.float32,
                            precision=lax.Precision.HIGHEST)
        w = w_ref[sl, :]
        wf = w.astype(jnp.float32)
        ssum = jnp.sum(s * wf, axis=1)
        smax = jnp.max(jnp.where(wf > 0, s, _NEG), axis=1)
        m_ref[0, 0, sl] = smax - ssum * (1.0 / L)


def _topk_body(m_ref, idx_ref, *, L, u, BH, PADU):
    m = m_ref[...]
    col = lax.broadcasted_iota(jnp.int32, (BH, L), 1)
    acc_col = lax.broadcasted_iota(jnp.int32, (BH, PADU), 1)

    def step(j, carry):
        m, acc = carry
        mx = jnp.max(m, axis=1, keepdims=True)
        amx = jnp.min(jnp.where(m == mx, col, L), axis=1,
                      keepdims=True)
        m = jnp.where(col == amx, _NEG, m)
        acc = jnp.where(acc_col == j, amx, acc)
        return m, acc

    _, acc = lax.fori_loop(0, u, step,
                           (m, jnp.zeros((BH, PADU), jnp.int32)))
    idx_ref[...] = acc


def _dense_body(idx_sref, q_ref, k_ref, v_ref, o_ref, *, L, D, u, scale):
    bh = pl.program_id(0)
    rows = [q_ref[0, pl.ds(idx_sref[bh, i], 1), :] for i in range(u)]
    qr = jnp.concatenate(rows, axis=0)
    k = k_ref[0]
    v = v_ref[0]
    s = lax.dot_general(qr, k, (((1,), (1,)), ((), ())),
                        preferred_element_type=jnp.float32,
                        precision=lax.Precision.HIGHEST) * scale
    smx = jnp.max(s, axis=1, keepdims=True)
    p = jnp.exp(s - smx)
    attn = p / jnp.sum(p, axis=1, keepdims=True)
    ctx = lax.dot_general(attn, v, (((1,), (0,)), ((), ())),
                          preferred_element_type=jnp.float32,
                          precision=lax.Precision.HIGHEST)
    vmean = jnp.mean(v, axis=0, keepdims=True)
    o_ref[0] = jnp.broadcast_to(vmean, (L, D))
    for i in range(u):
        o_ref[0, pl.ds(idx_sref[bh, i], 1), :] = ctx[i:i + 1, :]


def kernel(queries, keys, values):
    B, H, L_Q, D = queries.shape
    L_K = keys.shape[2]
    BH = B * H
    U_part = min(_FACTOR * int(np.ceil(np.log(L_K))), L_K)
    u = min(_FACTOR * int(np.ceil(np.log(L_Q))), L_Q)
    scale = 1.0 / sqrt(D)
    CH = 256
    PADU = 128

    q3 = queries.reshape(BH, L_Q, D)
    k3 = keys.reshape(BH, L_K, D)
    v3 = values.reshape(BH, L_K, D)
    w_cnt = jnp.asarray(_sample_count_matrix(L_Q, L_K, U_part))

    m_scores = pl.pallas_call(
        functools.partial(_m_scores_body, L=L_K, D=D, CH=CH),
        grid=(BH,),
        in_specs=[
            pl.BlockSpec((1, L_Q, D), lambda i: (i, 0, 0)),
            pl.BlockSpec((1, L_K, D), lambda i: (i, 0, 0)),
            pl.BlockSpec((L_Q, L_K), lambda i: (0, 0)),
        ],
        out_specs=pl.BlockSpec((1, 1, L_Q), lambda i: (i, 0, 0)),
        out_shape=jax.ShapeDtypeStruct((BH, 1, L_Q), jnp.float32),
    )(q3, k3, w_cnt)
    m_scores = m_scores.reshape(BH, L_Q)

    topk_idx = pl.pallas_call(
        functools.partial(_topk_body, L=L_Q, u=u, BH=BH, PADU=PADU),
        in_specs=[pl.BlockSpec((BH, L_Q), lambda: (0, 0))],
        out_specs=pl.BlockSpec((BH, PADU), lambda: (0, 0)),
        out_shape=jax.ShapeDtypeStruct((BH, PADU), jnp.int32),
    )(m_scores)

    grid_spec = pltpu.PrefetchScalarGridSpec(
        num_scalar_prefetch=1,
        grid=(BH,),
        in_specs=[
            pl.BlockSpec((1, L_Q, D), lambda i, *_: (i, 0, 0)),
            pl.BlockSpec((1, L_K, D), lambda i, *_: (i, 0, 0)),
            pl.BlockSpec((1, L_K, D), lambda i, *_: (i, 0, 0)),
        ],
        out_specs=pl.BlockSpec((1, L_Q, D), lambda i, *_: (i, 0, 0)),
    )
    out = pl.pallas_call(
        functools.partial(_dense_body, L=L_Q, D=D, u=u, scale=scale),
        grid_spec=grid_spec,
        out_shape=jax.ShapeDtypeStruct((BH, L_Q, D), jnp.float32),
    )(topk_idx, q3, k3, v3)

    return out.reshape(B, H, L_Q, D)


# speedup vs baseline: 1.7748x; 1.7748x over previous
import functools
from math import sqrt

import numpy as np
import jax
import jax.numpy as jnp
from jax import lax
from jax.experimental import pallas as pl
from jax.experimental.pallas import tpu as pltpu

_FACTOR = 5
_NEG = -3.0e38
_VALID_T = -1.0e37
_DELTA = 0.5
_NC = 128

_CONST_CACHE = {}

_ROT = ((13, 15, 26, 6), (17, 29, 16, 24))


def _rotl(x, r):
    return ((x << np.uint32(r)) | (x >> np.uint32(32 - r))).astype(np.uint32)


def _threefry2x32(k1, k2, x1, x2):
    with np.errstate(over="ignore"):
        ks = [np.uint32(k1), np.uint32(k2),
              np.uint32(k1) ^ np.uint32(k2) ^ np.uint32(0x1BD11BDA)]
        x = [(x1 + ks[0]).astype(np.uint32), (x2 + ks[1]).astype(np.uint32)]

        def rounds(x, rots):
            for r in rots:
                x[0] = (x[0] + x[1]).astype(np.uint32)
                x[1] = x[0] ^ _rotl(x[1], r)
            return x

        for i, sched in enumerate(((0, 1, 2), (1, 2, 0), (0, 0, 1),
                                   (1, 1, 2), (0, 2, 0))):
            x = rounds(x, _ROT[sched[0]])
            x[0] = (x[0] + ks[sched[1]]).astype(np.uint32)
            x[1] = (x[1] + ks[sched[2]] + np.uint32(i + 1)).astype(np.uint32)
    return x[0], x[1]


def _iota_2x32(n):
    i = np.arange(n, dtype=np.uint64)
    return ((i >> np.uint64(32)).astype(np.uint32),
            (i & np.uint64(0xFFFFFFFF)).astype(np.uint32))


def _np_randint_key42(shape, span):
    k1, k2 = np.uint32(0), np.uint32(42)
    hi, lo = _iota_2x32(2)
    b1, b2 = _threefry2x32(k1, k2, hi, lo)
    n = int(np.prod(shape))
    hi, lo = _iota_2x32(n)
    h1, h2 = _threefry2x32(b1[0], b2[0], hi, lo)
    l1, l2 = _threefry2x32(b1[1], b2[1], hi, lo)
    higher_bits, lower_bits = h1 ^ h2, l1 ^ l2
    span_u = np.uint32(span)
    mult = np.uint32((((2 ** 16) % span) ** 2) % span)
    with np.errstate(over="ignore"):
        off = ((higher_bits % span_u) * mult + (lower_bits % span_u)) % span_u
    return off.reshape(shape).astype(np.int64)


def _sample_count_matrix(L_Q, L_K, U_part):
    ck = (L_Q, L_K, U_part)
    if ck not in _CONST_CACHE:
        idx_np = _np_randint_key42((L_Q, U_part), L_K)
        w = np.zeros((L_Q, L_K), dtype=np.float32)
        np.add.at(w, (np.arange(L_Q)[:, None], idx_np), 1.0)
        _CONST_CACHE[ck] = w
    return _CONST_CACHE[ck]


def _split3(x):
    hi = x.astype(jnp.bfloat16)
    r1 = x - hi.astype(jnp.float32)
    mid = r1.astype(jnp.bfloat16)
    lo = (r1 - mid.astype(jnp.float32)).astype(jnp.bfloat16)
    return hi, mid, lo


def _dot_nt(a, b):
    return lax.dot_general(a, b, (((1,), (1,)), ((), ())),
                           preferred_element_type=jnp.float32)


def _m_approx_body(q_ref, k_ref, w_ref, m_ref, *, L, D, CH):
    k = k_ref[0, 0]
    kh = k.astype(jnp.bfloat16)
    for c in range(L // CH):
        sl = pl.ds(c * CH, CH)
        qh = q_ref[0, 0, sl, :].astype(jnp.bfloat16)
        s = _dot_nt(qh, kh)
        w = w_ref[sl, :]
        ks = lax.dot_general(w, kh, (((1,), (0,)), ((), ())),
                             preferred_element_type=jnp.float32)
        ssum = jnp.sum(q_ref[0, 0, sl, :] * ks, axis=1)
        smax = jnp.max(jnp.where(w > 0, s, _NEG), axis=1)
        m_ref[0, 0, sl] = smax - ssum * (1.0 / L)


def _cand_body(m_ref, idx_ref, val_ref, *, L, u, BH, NC):
    m0 = m_ref[:, 0, :]
    col = lax.broadcasted_iota(jnp.int32, (BH, L), 1)
    acc_col = lax.broadcasted_iota(jnp.int32, (BH, NC), 1)

    def tstep(j, m):
        mx = jnp.max(m, axis=1, keepdims=True)
        return jnp.where(m == mx, _NEG, m)

    m_rm = lax.fori_loop(0, u - 1, tstep, m0)
    thresh = jnp.max(m_rm, axis=1, keepdims=True) - _DELTA
    mc = jnp.where(m0 >= thresh, m0, _NEG)

    def cstep(j, carry):
        mc, acc, vac = carry
        mx = jnp.max(mc, axis=1, keepdims=True)
        qi = jnp.min(jnp.where(mc == mx, col, L), axis=1,
                     keepdims=True)
        valid = mx > _VALID_T
        mc = jnp.where(col == qi, _NEG, mc)
        acc = jnp.where(acc_col == j, jnp.where(valid, qi, 0), acc)
        vac = jnp.where(acc_col == j, jnp.where(valid, 1, 0), vac)
        return mc, acc, vac

    z = jnp.zeros((BH, NC), jnp.int32)
    _, acc, vac = lax.fori_loop(0, NC, cstep, (mc, z, z))
    idx_ref[...] = acc
    val_ref[:, 0, :] = vac


def _m_exact_body(idx_sref, q_ref, k_ref, w_ref, vld_ref, mex_ref,
                  qc_ref, wc_ref, sem, *, L, D, NC):
    bh = pl.program_id(0)
    cps = []
    for j in range(NC):
        qi = idx_sref[bh, j]
        cp = pltpu.make_async_copy(w_ref.at[pl.ds(qi, 1), :],
                                   wc_ref.at[pl.ds(j, 1), :], sem)
        cp.start()
        cps.append(cp)
        qc_ref[j:j + 1, :] = q_ref[0, 0, pl.ds(qi, 1), :]
    qc = qc_ref[...]
    k = k_ref[0, 0]
    kh, km, kl = _split3(k)
    qh, qm, ql = _split3(qc)
    s = (_dot_nt(qh, kh) + _dot_nt(qh, km) + _dot_nt(qm, kh)
         + _dot_nt(qh, kl) + _dot_nt(qm, km) + _dot_nt(ql, kh))
    for cp in cps:
        cp.wait()
    wf = wc_ref[...]
    ssum = jnp.sum(s * wf, axis=1)
    smax = jnp.max(jnp.where(wf > 0, s, _NEG), axis=1)
    mex = smax - ssum * (1.0 / L)
    valid = vld_ref[0, 0, :]
    mex_ref[0, 0, :] = jnp.where(valid > 0, mex, _NEG)


def _final_sel_body(mex_ref, cidx_ref, idx_ref, *, u, BH, NC, PADU):
    mex = mex_ref[:, 0, :]
    cidx = cidx_ref[...]
    slot = lax.broadcasted_iota(jnp.int32, (BH, NC), 1)
    acc_col = lax.broadcasted_iota(jnp.int32, (BH, PADU), 1)

    def step(j, carry):
        mex, acc = carry
        mx = jnp.max(mex, axis=1, keepdims=True)
        si = jnp.min(jnp.where(mex == mx, slot, NC), axis=1, keepdims=True)
        qi = jnp.min(jnp.where(slot == si, cidx, 2 ** 30), axis=1,
                     keepdims=True)
        mex = jnp.where(slot == si, _NEG, mex)
        acc = jnp.where(acc_col == j, qi, acc)
        return mex, acc

    _, acc = lax.fori_loop(0, u, step, (mex, jnp.zeros((BH, PADU), jnp.int32)))
    idx_ref[...] = acc


def _dense_body(idx_sref, q_ref, k_ref, v_ref, o_ref, *, L, D, u, scale):
    bh = pl.program_id(0)
    rows = [q_ref[0, 0, pl.ds(idx_sref[bh, i], 1), :] for i in range(u)]
    qr = jnp.concatenate(rows, axis=0)
    k = k_ref[0, 0]
    v = v_ref[0, 0]
    s = lax.dot_general(qr, k, (((1,), (1,)), ((), ())),
                        preferred_element_type=jnp.float32,
                        precision=lax.Precision.HIGHEST) * scale
    smx = jnp.max(s, axis=1, keepdims=True)
    p = jnp.exp(s - smx)
    attn = p / jnp.sum(p, axis=1, keepdims=True)
    ctx = lax.dot_general(attn, v, (((1,), (0,)), ((), ())),
                          preferred_element_type=jnp.float32,
                          precision=lax.Precision.HIGHEST)
    vmean = jnp.mean(v, axis=0, keepdims=True)
    o_ref[0, 0] = jnp.broadcast_to(vmean, (L, D))
    for i in range(u):
        o_ref[0, 0, pl.ds(idx_sref[bh, i], 1), :] = ctx[i:i + 1, :]


def kernel(queries, keys, values):
    B, H, L_Q, D = queries.shape
    L_K = keys.shape[2]
    BH = B * H
    U_part = min(_FACTOR * int(np.ceil(np.log(L_K))), L_K)
    u = min(_FACTOR * int(np.ceil(np.log(L_Q))), L_Q)
    scale = 1.0 / sqrt(D)
    CH = 256
    PADU = 128
    NC = _NC

    w_np = _sample_count_matrix(L_Q, L_K, U_part)
    w_f32 = jnp.asarray(w_np)
    w_cnt = w_f32.astype(jnp.bfloat16)

    def bh_map(i):
        return (i // H, i % H, 0, 0)

    qkv_spec = pl.BlockSpec((1, 1, L_Q, D), bh_map)

    m_approx = pl.pallas_call(
        functools.partial(_m_approx_body, L=L_K, D=D, CH=CH),
        grid=(BH,),
        in_specs=[qkv_spec, qkv_spec,
                  pl.BlockSpec((L_Q, L_K), lambda i: (0, 0))],
        out_specs=pl.BlockSpec((1, 1, L_Q), lambda i: (i, 0, 0)),
        out_shape=jax.ShapeDtypeStruct((BH, 1, L_Q), jnp.float32),
    )(queries, keys, w_cnt)

    cand_idx, cand_valid = pl.pallas_call(
        functools.partial(_cand_body, L=L_Q, u=u, BH=BH, NC=NC),
        in_specs=[pl.BlockSpec((BH, 1, L_Q), lambda: (0, 0, 0))],
        out_specs=[pl.BlockSpec((BH, NC), lambda: (0, 0)),
                   pl.BlockSpec((BH, 1, NC), lambda: (0, 0, 0))],
        out_shape=[jax.ShapeDtypeStruct((BH, NC), jnp.int32),
                   jax.ShapeDtypeStruct((BH, 1, NC), jnp.int32)],
    )(m_approx)

    grid_a2 = pltpu.PrefetchScalarGridSpec(
        num_scalar_prefetch=1,
        grid=(BH,),
        in_specs=[
            pl.BlockSpec((1, 1, L_Q, D), lambda i, *_: bh_map(i)),
            pl.BlockSpec((1, 1, L_K, D), lambda i, *_: bh_map(i)),
            pl.BlockSpec(memory_space=pl.ANY),
            pl.BlockSpec((1, 1, NC), lambda i, *_: (i, 0, 0)),
        ],
        out_specs=pl.BlockSpec((1, 1, NC), lambda i, *_: (i, 0, 0)),
        scratch_shapes=[pltpu.VMEM((NC, D), jnp.float32),
                        pltpu.VMEM((NC, L_K), jnp.float32),
                        pltpu.SemaphoreType.DMA],
    )
    m_exact = pl.pallas_call(
        functools.partial(_m_exact_body, L=L_K, D=D, NC=NC),
        grid_spec=grid_a2,
        out_shape=jax.ShapeDtypeStruct((BH, 1, NC), jnp.float32),
    )(cand_idx, queries, keys, w_f32, cand_valid)

    topk_idx = pl.pallas_call(
        functools.partial(_final_sel_body, u=u, BH=BH, NC=NC, PADU=PADU),
        in_specs=[pl.BlockSpec((BH, 1, NC), lambda: (0, 0, 0)),
                  pl.BlockSpec((BH, NC), lambda: (0, 0))],
        out_specs=pl.BlockSpec((BH, PADU), lambda: (0, 0)),
        out_shape=jax.ShapeDtypeStruct((BH, PADU), jnp.int32),
    )(m_exact, cand_idx)

    grid_c = pltpu.PrefetchScalarGridSpec(
        num_scalar_prefetch=1,
        grid=(BH,),
        in_specs=[
            pl.BlockSpec((1, 1, L_Q, D), lambda i, *_: bh_map(i)),
            pl.BlockSpec((1, 1, L_K, D), lambda i, *_: bh_map(i)),
            pl.BlockSpec((1, 1, L_K, D), lambda i, *_: bh_map(i)),
        ],
        out_specs=pl.BlockSpec((1, 1, L_Q, D), lambda i, *_: bh_map(i)),
    )
    out = pl.pallas_call(
        functools.partial(_dense_body, L=L_Q, D=D, u=u, scale=scale),
        grid_spec=grid_c,
        out_shape=jax.ShapeDtypeStruct((B, H, L_Q, D), jnp.float32),
    )(topk_idx, queries, keys, values)

    return out
